# baseline (speedup 1.0000x reference)
import functools

import jax
import jax.numpy as jnp
from jax import lax
from jax.experimental import pallas as pl
from jax.experimental.pallas import tpu as pltpu
from jax.experimental.pallas import tpu_sc as plsc

B, F, L, V, D = 1024, 4, 50, 100000, 128
NC, NS, LANES = 2, 16, 16
NW = NC * NS
PAIRS = F * B
PPW = PAIRS // NW
CP = 2
NCHUNK = PPW // CP
CIDX = CP * L
CPAD = 104
NSUB = D // LANES
NBUF = 2


def _sc_body(x_hbm, e0, e1, e2, e3, out_hbm, idx_v, buf_v, out_v,
             sem0, sem1, sem2, sem3):
  cid = lax.axis_index("c")
  sid = lax.axis_index("s")
  wid = sid * NC + cid
  f = wid // (NW // F)

  pltpu.sync_copy(x_hbm.at[wid], idx_v)

  def compute(buf, j):
    for p in range(CP):
      def l_body(l, acc):
        row = p * L + l
        new = list(acc)
        for k in range(NSUB):
          v = buf[row, pl.ds(k * LANES, LANES)]
          new[k] = acc[k] + v
          new[NSUB + k] = acc[NSUB + k] + jnp.where(
              v != 0.0, jnp.float32(1.0), jnp.float32(0.0))
        return tuple(new)

      zeros = tuple(jnp.zeros((LANES,), jnp.float32) for _ in range(2 * NSUB))
      acc = lax.fori_loop(0, L, l_body, zeros)
      orow = j * CP + p
      for k in range(NSUB):
        out_v[orow, pl.ds(k * LANES, LANES)] = (
            acc[k] / (acc[NSUB + k] + jnp.float32(1e-16)))

  sems = (sem0, sem1, sem2, sem3)

  def process(table):
    for b in range(NBUF - 1):
      pltpu.async_copy(table.at[idx_v.at[b, pl.ds(0, CIDX)]], buf_v.at[b],
                       sems[b])

    def outer(i, carry):
      j0 = NBUF * i
      for b in range(NBUF):
        j = j0 + b
        pltpu.make_async_copy(table.at[idx_v.at[j, pl.ds(0, CIDX)]],
                              buf_v.at[b], sems[b]).wait()
        compute(buf_v.at[b], j)
        nj = j + NBUF - 1
        sl = (b + NBUF - 1) % NBUF

        @pl.when(nj < NCHUNK)
        def _(nj=nj, sl=sl):
          pltpu.async_copy(table.at[idx_v.at[nj, pl.ds(0, CIDX)]],
                           buf_v.at[sl], sems[sl])
      return carry

    lax.fori_loop(0, NCHUNK // NBUF, outer, 0)

  @pl.when(f == 0)
  def _():
    process(e0)

  @pl.when(f == 1)
  def _():
    process(e1)

  @pl.when(f == 2)
  def _():
    process(e2)

  @pl.when(f == 3)
  def _():
    process(e3)

  pltpu.sync_copy(out_v, out_hbm.at[wid])


@jax.jit
def kernel(x, emb0, emb1, emb2, emb3):
  xt = jnp.transpose(x, (1, 0, 2)).reshape(NW, NCHUNK, CIDX)
  xpad = jnp.pad(xt, ((0, 0), (0, 0), (0, CPAD - CIDX)))

  mesh = plsc.VectorSubcoreMesh(core_axis_name="c", subcore_axis_name="s")
  out = pl.kernel(
      _sc_body,
      out_type=jax.ShapeDtypeStruct((NW, PPW, D), jnp.float32),
      mesh=mesh,
      scratch_types=[
          pltpu.VMEM((NCHUNK, CPAD), jnp.int32),
          pltpu.VMEM((NBUF, CIDX, D), jnp.float32),
          pltpu.VMEM((PPW, D), jnp.float32),
          pltpu.SemaphoreType.DMA,
          pltpu.SemaphoreType.DMA,
          pltpu.SemaphoreType.DMA,
          pltpu.SemaphoreType.DMA,
      ],
  )(xpad, emb0, emb1, emb2, emb3)

  return out.reshape(F, B, D).transpose(1, 0, 2)


# speedup vs baseline: 1.1292x; 1.1292x over previous
import functools

import jax
import jax.numpy as jnp
from jax import lax
from jax.experimental import pallas as pl
from jax.experimental.pallas import tpu as pltpu
from jax.experimental.pallas import tpu_sc as plsc

B, F, L, V, D = 1024, 4, 50, 100000, 128
NC, NS, LANES = 2, 16, 16
NW = NC * NS
PAIRS = F * B
PPW = PAIRS // NW
CP = 2
NCHUNK = PPW // CP
CIDX = CP * L
CPAD = 104
NSUB = D // LANES


def _sc_body(x_hbm, e0, e1, e2, e3, out_hbm, idx_v, buf_v, out_v, sem0, sem1):
  cid = lax.axis_index("c")
  sid = lax.axis_index("s")
  wid = sid * NC + cid
  f = wid // (NW // F)

  pltpu.sync_copy(x_hbm.at[wid], idx_v)

  def compute(buf, j):
    for p in range(CP):
      def l_body(l, acc):
        row = p * L + l
        new = list(acc)
        for k in range(NSUB):
          v = buf[row, pl.ds(k * LANES, LANES)]
          new[k] = acc[k] + v
          new[NSUB + k] = acc[NSUB + k] + jnp.where(
              v != 0.0, jnp.float32(1.0), jnp.float32(0.0))
        return tuple(new)

      zeros = tuple(jnp.zeros((LANES,), jnp.float32) for _ in range(2 * NSUB))
      acc = lax.fori_loop(0, L, l_body, zeros)
      orow = j * CP + p
      for k in range(NSUB):
        out_v[orow, pl.ds(k * LANES, LANES)] = (
            acc[k] / (acc[NSUB + k] + jnp.float32(1e-16)))

  def process(table):
    pltpu.async_copy(table.at[idx_v.at[0]], buf_v.at[0], sem0)

    def outer(i, carry):
      j0 = 2 * i
      pltpu.async_copy(table.at[idx_v.at[j0 + 1]], buf_v.at[1], sem1)
      pltpu.make_async_copy(table.at[idx_v.at[j0]], buf_v.at[0], sem0).wait()
      compute(buf_v.at[0], j0)

      @pl.when(i + 1 < NCHUNK // 2)
      def _():
        pltpu.async_copy(table.at[idx_v.at[j0 + 2]], buf_v.at[0], sem0)

      pltpu.make_async_copy(
          table.at[idx_v.at[j0 + 1]], buf_v.at[1], sem1).wait()
      compute(buf_v.at[1], j0 + 1)
      return carry

    lax.fori_loop(0, NCHUNK // 2, outer, 0)

  @pl.when(f == 0)
  def _():
    process(e0)

  @pl.when(f == 1)
  def _():
    process(e1)

  @pl.when(f == 2)
  def _():
    process(e2)

  @pl.when(f == 3)
  def _():
    process(e3)

  b0 = (wid % (NW // F)) * PPW
  pltpu.sync_copy(out_v, out_hbm.at[pl.ds(b0, PPW), f])


@jax.jit
def kernel(x, emb0, emb1, emb2, emb3):
  xt = jnp.transpose(x, (1, 0, 2)).reshape(NW, NCHUNK, CIDX)
  xpad = jnp.pad(xt, ((0, 0), (0, 0), (0, CPAD - CIDX)))

  mesh = plsc.VectorSubcoreMesh(core_axis_name="c", subcore_axis_name="s")
  out = pl.kernel(
      _sc_body,
      out_type=jax.ShapeDtypeStruct((B, F, D), jnp.float32),
      mesh=mesh,
      scratch_types=[
          pltpu.VMEM((NCHUNK, CPAD), jnp.int32),
          pltpu.VMEM((2, CPAD, D), jnp.float32),
          pltpu.VMEM((PPW, D), jnp.float32),
          pltpu.SemaphoreType.DMA,
          pltpu.SemaphoreType.DMA,
      ],
  )(xpad, emb0, emb1, emb2, emb3)

  return out
